# baseline (speedup 1.0000x reference)
import functools

import jax
import jax.numpy as jnp
from jax import lax
from jax.experimental import pallas as pl
from jax.experimental.pallas import tpu as pltpu
from jax.experimental.pallas import tpu_sc as plsc


def _sc_gather_last(idx, table):
    B, = idx.shape
    V, D = table.shape
    info = plsc.get_sparse_core_info()
    NC, NS = 1, info.num_subcores
    NW = NC * NS
    b_per_w = B // NW

    mesh = plsc.VectorSubcoreMesh(
        core_axis_name="c", subcore_axis_name="s", num_cores=NC
    )

    @functools.partial(
        pl.kernel,
        mesh=mesh,
        out_type=jax.ShapeDtypeStruct((B, D), jnp.float32),
        scratch_types=[
            pltpu.VMEM((b_per_w,), jnp.int32),
            pltpu.VMEM((b_per_w, D), jnp.float32),
            pltpu.SemaphoreType.DMA,
        ],
        compiler_params=pltpu.CompilerParams(
            use_tc_tiling_on_sc=False, skip_device_barrier=True
        ),
    )
    def k(idx_hbm, table_hbm, out_hbm, idx_v, rows_v, sem):
        wid = lax.axis_index("s") * NC + lax.axis_index("c")
        base = wid * b_per_w
        pltpu.sync_copy(idx_hbm.at[pl.ds(base, b_per_w)], idx_v)
        pltpu.async_copy(table_hbm.at[idx_v], rows_v, sem).wait()
        pltpu.sync_copy(rows_v, out_hbm.at[pl.ds(base, b_per_w)])

    return k(idx, table)


def _tc_dense(z, Wt, b2):
    B, D = z.shape
    BLK = 2048

    def body(z_ref, w_ref, b_ref, o_ref):
        acc = jnp.dot(z_ref[...], w_ref[...], preferred_element_type=jnp.float32)
        o_ref[...] = jnp.tanh(acc + b_ref[...])

    return pl.pallas_call(
        body,
        grid=(B // BLK,),
        in_specs=[
            pl.BlockSpec((BLK, D), lambda i: (i, 0)),
            pl.BlockSpec((D, D), lambda i: (0, 0)),
            pl.BlockSpec((1, D), lambda i: (0, 0)),
        ],
        out_specs=pl.BlockSpec((BLK, D), lambda i: (i, 0)),
        out_shape=jax.ShapeDtypeStruct((B, D), jnp.float32),
    )(z, Wt, b2)


def kernel(x, table, W, b):
    z = _sc_gather_last(x[:, -1].astype(jnp.int32), table)
    return _tc_dense(z, W.T, b.reshape(1, -1))


# speedup vs baseline: 2.4113x; 2.4113x over previous
import functools

import jax
import jax.numpy as jnp
from jax import lax
from jax.experimental import pallas as pl
from jax.experimental.pallas import tpu as pltpu
from jax.experimental.pallas import tpu_sc as plsc

_BLK = 8192


def _tc_premul(table_t, W):
    C, R = table_t.shape
    grid = (R // 2 + _BLK - 1) // _BLK
    H = grid * _BLK

    def body(t1_ref, t2_ref, w_ref, o_ref):
        dn = (((0,), (1,)), ((), ()))
        acc1 = jax.lax.dot_general(
            t1_ref[...], w_ref[...], dn, preferred_element_type=jnp.float32
        )
        acc2 = jax.lax.dot_general(
            t2_ref[...], w_ref[...], dn, preferred_element_type=jnp.float32
        )
        o_ref[...] = jnp.concatenate([acc1, acc2], axis=1)

    nb = grid
    last_blk = (R - 1) // _BLK

    return pl.pallas_call(
        body,
        grid=(grid,),
        in_specs=[
            pl.BlockSpec((C, _BLK), lambda i: (0, i)),
            pl.BlockSpec((C, _BLK), lambda i: (0, jnp.minimum(i + nb, last_blk))),
            pl.BlockSpec((C, C), lambda i: (0, 0)),
        ],
        out_specs=pl.BlockSpec((_BLK, 2 * C), lambda i: (i, 0)),
        out_shape=jax.ShapeDtypeStruct((H, 2 * C), jnp.float32),
    )(table_t, table_t, W)


def _sc_gather(idx, mm_flat):
    B, = idx.shape
    N, D = mm_flat.shape
    info = plsc.get_sparse_core_info()
    NC, NS = info.num_cores, info.num_subcores
    NW = NC * NS
    b_per_w = B // NW

    mesh = plsc.VectorSubcoreMesh(core_axis_name="c", subcore_axis_name="s")

    @functools.partial(
        pl.kernel,
        mesh=mesh,
        out_type=jax.ShapeDtypeStruct((B, D), jnp.float32),
        scratch_types=[
            pltpu.VMEM((b_per_w,), jnp.int32),
            pltpu.VMEM((b_per_w, D), jnp.float32),
            pltpu.SemaphoreType.DMA,
        ],
        compiler_params=pltpu.CompilerParams(use_tc_tiling_on_sc=False),
    )
    def k(idx_hbm, mm_hbm, out_hbm, idx_v, rows_v, sem):
        wid = lax.axis_index("s") * NC + lax.axis_index("c")
        base = wid * b_per_w
        pltpu.sync_copy(idx_hbm.at[pl.ds(base, b_per_w)], idx_v)
        pltpu.async_copy(mm_hbm.at[idx_v], rows_v, sem).wait()
        pltpu.sync_copy(rows_v, out_hbm.at[pl.ds(base, b_per_w)])

    return k(idx, mm_flat)


def _tc_bias_tanh(zp, b128):
    N, D2 = zp.shape
    BLK = 2048

    def body(z_ref, b_ref, o_ref):
        o_ref[...] = jnp.tanh(z_ref[...] + b_ref[...])

    return pl.pallas_call(
        body,
        grid=(N // BLK,),
        in_specs=[
            pl.BlockSpec((BLK, D2), lambda i: (i, 0)),
            pl.BlockSpec((1, D2), lambda i: (0, 0)),
        ],
        out_specs=pl.BlockSpec((BLK, D2), lambda i: (i, 0)),
        out_shape=jax.ShapeDtypeStruct((N, D2), jnp.float32),
    )(zp, b128)


def kernel(x, table, W, b):
    B = x.shape[0]
    R, D = table.shape
    mm = _tc_premul(table.T, W)
    H = mm.shape[0]
    last = x[:, -1].astype(jnp.int32)
    fidx = jnp.where(last < H, 2 * last, 2 * (last - H) + 1)
    mm_flat = mm.reshape(2 * H, D)
    z = _sc_gather(fidx, mm_flat)
    zp = z.reshape(B // 2, 2 * D)
    b128 = jnp.concatenate([b, b]).reshape(1, 2 * D)
    out = _tc_bias_tanh(zp, b128)
    return out.reshape(B, D)


# speedup vs baseline: 2.5561x; 1.0601x over previous
import functools

import jax
import jax.numpy as jnp
from jax import lax
from jax.experimental import pallas as pl
from jax.experimental.pallas import tpu as pltpu
from jax.experimental.pallas import tpu_sc as plsc

_BLK = 16384


def _tc_premul(table_t, W):
    C, R = table_t.shape
    grid = (R // 2 + _BLK - 1) // _BLK
    H = grid * _BLK

    def body(t1_ref, t2_ref, w_ref, o_ref):
        dn = (((0,), (1,)), ((), ()))
        acc1 = jax.lax.dot_general(
            t1_ref[...], w_ref[...], dn, preferred_element_type=jnp.float32
        )
        acc2 = jax.lax.dot_general(
            t2_ref[...], w_ref[...], dn, preferred_element_type=jnp.float32
        )
        o_ref[...] = jnp.concatenate([acc1, acc2], axis=1)

    nb = grid
    last_blk = (R - 1) // _BLK

    return pl.pallas_call(
        body,
        grid=(grid,),
        in_specs=[
            pl.BlockSpec((C, _BLK), lambda i: (0, i)),
            pl.BlockSpec((C, _BLK), lambda i: (0, jnp.minimum(i + nb, last_blk))),
            pl.BlockSpec((C, C), lambda i: (0, 0)),
        ],
        out_specs=pl.BlockSpec((_BLK, 2 * C), lambda i: (i, 0)),
        out_shape=jax.ShapeDtypeStruct((H, 2 * C), jnp.float32),
    )(table_t, table_t, W)


def _sc_gather(idx, mm_flat):
    B, = idx.shape
    N, D = mm_flat.shape
    info = plsc.get_sparse_core_info()
    NC, NS = info.num_cores, info.num_subcores
    NW = NC * NS
    b_per_w = B // NW

    mesh = plsc.VectorSubcoreMesh(core_axis_name="c", subcore_axis_name="s")

    @functools.partial(
        pl.kernel,
        mesh=mesh,
        out_type=jax.ShapeDtypeStruct((B, D), jnp.float32),
        scratch_types=[
            pltpu.VMEM((b_per_w,), jnp.int32),
            pltpu.VMEM((b_per_w, D), jnp.float32),
            pltpu.SemaphoreType.DMA,
        ],
        compiler_params=pltpu.CompilerParams(use_tc_tiling_on_sc=False),
    )
    def k(idx_hbm, mm_hbm, out_hbm, idx_v, rows_v, sem):
        wid = lax.axis_index("s") * NC + lax.axis_index("c")
        base = wid * b_per_w
        pltpu.sync_copy(idx_hbm.at[pl.ds(base, b_per_w)], idx_v)
        pltpu.async_copy(mm_hbm.at[idx_v], rows_v, sem).wait()
        pltpu.sync_copy(rows_v, out_hbm.at[pl.ds(base, b_per_w)])

    return k(idx, mm_flat)


def _tc_bias_tanh(zp, b128):
    N, D2 = zp.shape
    BLK = 2048

    def body(z_ref, b_ref, o_ref):
        o_ref[...] = jnp.tanh(z_ref[...] + b_ref[...])

    return pl.pallas_call(
        body,
        grid=(N // BLK,),
        in_specs=[
            pl.BlockSpec((BLK, D2), lambda i: (i, 0)),
            pl.BlockSpec((1, D2), lambda i: (0, 0)),
        ],
        out_specs=pl.BlockSpec((BLK, D2), lambda i: (i, 0)),
        out_shape=jax.ShapeDtypeStruct((N, D2), jnp.float32),
    )(zp, b128)


def kernel(x, table, W, b):
    B = x.shape[0]
    R, D = table.shape
    mm = _tc_premul(table.T, W)
    H = mm.shape[0]
    last = x[:, -1].astype(jnp.int32)
    fidx = jnp.where(last < H, 2 * last, 2 * (last - H) + 1)
    mm_flat = mm.reshape(2 * H, D)
    z = _sc_gather(fidx, mm_flat)
    zp = z.reshape(B // 2, 2 * D)
    b128 = jnp.concatenate([b, b]).reshape(1, 2 * D)
    out = _tc_bias_tanh(zp, b128)
    return out.reshape(B, D)
